# Initial kernel scaffold; baseline (speedup 1.0000x reference)
import jax
import jax.numpy as jnp
from jax.experimental import pallas as pl


def kernel(inp, lut, bias):
    raise NotImplementedError("write your pallas kernel here")


# speedup vs baseline: 2.9340x; 2.9340x over previous
import functools

import jax
import jax.numpy as jnp
from jax import lax
from jax.experimental import pallas as pl
from jax.experimental.pallas import tpu as pltpu
from jax.experimental.pallas import tpu_sc as plsc

IN_DIM = 1000000
HID = 32
BATCH = 16384
HIST = 50

NC = 2
NS = 16
NW = NC * NS
LANES = 16

ROWS_PER_W = BATCH // NW
CHUNK = 32
NCHUNK = ROWS_PER_W // CHUNK
IDX_PER_CHUNK = CHUNK * HIST
SUB = 20
IDX_PER_SUB = IDX_PER_CHUNK // SUB
NACC = 4


def _encoder_body(inp_hbm, lut_hbm, bias_hbm, out_hbm,
                  idx0_v, idx1_v, rows0_v, rows1_v, out_v, bias_v,
                  sem0, sem1):
    wid = lax.axis_index("c") * NS + lax.axis_index("s")
    wrow0 = wid * ROWS_PER_W
    widx0 = wrow0 * HIST

    pltpu.sync_copy(bias_hbm, bias_v)
    bias_lo = bias_v[pl.ds(0, LANES)]
    bias_hi = bias_v[pl.ds(LANES, LANES)]

    def stage(c, idx_ref, rows_ref, sem):
        pltpu.sync_copy(inp_hbm.at[pl.ds(widx0 + c * IDX_PER_CHUNK,
                                         IDX_PER_CHUNK)], idx_ref)
        for j in range(SUB):
            pltpu.async_copy(
                lut_hbm.at[idx_ref.at[pl.ds(j * IDX_PER_SUB, IDX_PER_SUB)]],
                rows_ref.at[pl.ds(j * IDX_PER_SUB, IDX_PER_SUB)],
                sem)

    def consume(c, rows_ref, sem):
        pltpu.make_async_copy(lut_hbm.at[pl.ds(0, IDX_PER_CHUNK)],
                              rows_ref, sem).wait()

        def body(b, carry):
            base = b * HIST
            acc = [None] * (2 * NACC)
            for l in range(HIST):
                for h in range(2):
                    v = rows_ref[base + l, pl.ds(h * LANES, LANES)]
                    k = h * NACC + (l % NACC)
                    acc[k] = v if acc[k] is None else acc[k] + v
            lo = (acc[0] + acc[1]) + (acc[2] + acc[3]) + bias_lo
            hi = (acc[4] + acc[5]) + (acc[6] + acc[7]) + bias_hi
            out_v[b, pl.ds(0, LANES)] = lo
            out_v[b, pl.ds(LANES, LANES)] = hi
            return carry

        lax.fori_loop(0, CHUNK, body, 0)
        pltpu.sync_copy(out_v, out_hbm.at[pl.ds(wrow0 + c * CHUNK, CHUNK)])

    stage(0, idx0_v, rows0_v, sem0)

    def pair_body(p, carry):
        c0 = 2 * p
        stage(c0 + 1, idx1_v, rows1_v, sem1)
        consume(c0, rows0_v, sem0)

        @pl.when(c0 + 2 < NCHUNK)
        def _():
            stage(c0 + 2, idx0_v, rows0_v, sem0)

        consume(c0 + 1, rows1_v, sem1)
        return carry

    lax.fori_loop(0, NCHUNK // 2, pair_body, 0)


_encoder = functools.partial(
    pl.kernel,
    out_type=jax.ShapeDtypeStruct((BATCH, HID), jnp.float32),
    mesh=plsc.VectorSubcoreMesh(core_axis_name="c", subcore_axis_name="s",
                                num_cores=NC, num_subcores=NS),
    scratch_types=[
        pltpu.VMEM((IDX_PER_CHUNK,), jnp.int32),
        pltpu.VMEM((IDX_PER_CHUNK,), jnp.int32),
        pltpu.VMEM((IDX_PER_CHUNK, HID), jnp.float32),
        pltpu.VMEM((IDX_PER_CHUNK, HID), jnp.float32),
        pltpu.VMEM((CHUNK, HID), jnp.float32),
        pltpu.VMEM((HID,), jnp.float32),
        pltpu.SemaphoreType.DMA,
        pltpu.SemaphoreType.DMA,
    ],
    compiler_params=pltpu.CompilerParams(use_tc_tiling_on_sc=False),
)(_encoder_body)


def kernel(inp, lut, bias):
    inp_flat = inp.reshape(-1).astype(jnp.int32)
    return _encoder(inp_flat, lut, bias)
